# Initial kernel scaffold; baseline (speedup 1.0000x reference)
import jax
import jax.numpy as jnp
from jax.experimental import pallas as pl


def kernel(pred_tensor, target_tensor):
    raise NotImplementedError("write your pallas kernel here")


# speedup vs baseline: 3.5494x; 3.5494x over previous
import functools
import jax
import jax.numpy as jnp
from jax import lax
from jax.experimental import pallas as pl
from jax.experimental.pallas import tpu as pltpu
from jax.experimental.pallas import tpu_sc as plsc

BATCH = 1024
S = 7
N = 20
CELLS = BATCH * S * S
NC = 2
NS = 16
NW = NC * NS
CPT = CELLS // NW
GROUPS = CPT // 16
WPT = CPT * N
Sf = 7.0


def _sq(x):
    return x * x


def _sqrt16(x):
    xi = plsc.bitcast(x, jnp.int32)
    yi = jnp.int32(0x5F3759DF) - lax.shift_right_arithmetic(xi, 1)
    y = plsc.bitcast(yi, jnp.float32)
    y = y * (1.5 - 0.5 * x * y * y)
    y = y * (1.5 - 0.5 * x * y * y)
    y = y * (1.5 - 0.5 * x * y * y)
    return jnp.where(x == 0.0, 0.0, x * y)


def _body(pred_hbm, targ_hbm, out_hbm, pred_v, targ_v, acc_v):
    wid = lax.axis_index("s") * NC + lax.axis_index("c")
    base = wid * WPT
    pltpu.sync_copy(pred_hbm.at[pl.ds(base, WPT)], pred_v)
    pltpu.sync_copy(targ_hbm.at[pl.ds(base, WPT)], targ_v)
    lanes = lax.iota(jnp.int32, 16) * N

    def group(g, acc):
        col0 = g * (16 * N) + lanes

        def pch(c):
            return plsc.load_gather(pred_v, [col0 + c])

        def tch(c):
            return plsc.load_gather(targ_v, [col0 + c])

        p = [pch(c) for c in range(10)]
        t = [tch(c) for c in range(10)]
        t4 = t[4]
        m = jnp.where(t4 > 0.0, 1.0, 0.0)
        l_noobj = jnp.where(t4 == 0.0,
                            _sq(p[4] - t4) + _sq(p[9] - t[9]),
                            0.0)
        l_class = _sq(pch(10) - tch(10))
        for c in range(11, 20):
            l_class = l_class + _sq(pch(c) - tch(c))
        lt_t0 = t[2] / Sf - 0.5 * t[2]
        lt_t1 = t[2] / Sf - 0.5 * t[3]
        rb_t0 = t[2] / Sf + 0.5 * t[2]
        rb_t1 = t[2] / Sf + 0.5 * t[3]
        area2 = (rb_t0 - lt_t0) * (rb_t1 - lt_t1)
        ious = []
        for b in (0, 1):
            lt0 = p[2] / Sf - 0.5 * p[5 * b + 2]
            lt1 = p[7] / Sf - 0.5 * p[5 * b + 3]
            rb0 = p[2] / Sf + 0.5 * p[5 * b + 2]
            rb1 = p[7] / Sf + 0.5 * p[5 * b + 3]
            w = jnp.maximum(jnp.minimum(rb0, rb_t0) - jnp.maximum(lt0, lt_t0), 0.0)
            h = jnp.maximum(jnp.minimum(rb1, rb_t1) - jnp.maximum(lt1, lt_t1), 0.0)
            inter = w * h
            area1 = (rb0 - lt0) * (rb1 - lt1)
            ious.append(inter / (area1 + area2 - inter))
        sel = ious[0] >= ious[1]
        max_iou = jnp.maximum(ious[0], ious[1])
        pr = [jnp.where(sel, p[j], p[5 + j]) for j in range(5)]
        tr = [jnp.where(sel, t[j], t[5 + j]) for j in range(4)]
        l_xy = _sq(pr[0] - tr[0]) + _sq(pr[1] - tr[1])
        l_wh = _sq(_sqrt16(pr[2]) - _sqrt16(tr[2])) \
            + _sq(_sqrt16(pr[3]) - _sqrt16(tr[3]))
        l_obj = _sq(pr[4] - max_iou)
        return acc + (m * (5.0 * (l_xy + l_wh) + l_obj)
                      + l_class * m + 0.5 * l_noobj)

    acc = lax.fori_loop(0, GROUPS, group, jnp.zeros((16,), jnp.float32))
    acc_v[...] = acc
    pltpu.sync_copy(acc_v, out_hbm.at[wid])


@jax.jit
def _yolo_sc(pred_flat, targ_flat):
    mesh = plsc.VectorSubcoreMesh(
        core_axis_name="c", subcore_axis_name="s",
        num_cores=NC, num_subcores=NS)
    run = pl.kernel(
        _body,
        out_type=jax.ShapeDtypeStruct((NW, 16), jnp.float32),
        mesh=mesh,
        scratch_types=[
            pltpu.VMEM((WPT,), jnp.float32),
            pltpu.VMEM((WPT,), jnp.float32),
            pltpu.VMEM((16,), jnp.float32),
        ],
        compiler_params=pltpu.CompilerParams(needs_layout_passes=False),
    )
    partials = run(pred_flat, targ_flat)
    return jnp.sum(partials) * (1.0 / BATCH)


def kernel(pred_tensor, target_tensor):
    return _yolo_sc(pred_tensor.reshape(-1), target_tensor.reshape(-1))
